# initial kernel scaffold (unmeasured)
import jax
import jax.numpy as jnp
from jax import lax
from jax.experimental import pallas as pl
from jax.experimental.pallas import tpu as pltpu

N_DEV = 8
M_PER = 512
K = 4096
N_PER = 256


def _ring(p):
    return jnp.where(p < 4, p, 11 - p)


def kernel(x, w_mat):
    x16 = x.astype(jnp.bfloat16)
    w16 = w_mat.astype(jnp.bfloat16)

    def body(x_ref, w_ref, out_ref, gather, amax_src, amax_buf,
             send_sems, recv_sems, a_send_sems, a_recv_sems):
        me = lax.axis_index("i")
        r = _ring(me)
        right = _ring((r + 1) % N_DEV)
        left = _ring((r - 1) % N_DEV)

        barrier = pltpu.get_barrier_semaphore()
        for nbr in (left, right):
            pl.semaphore_signal(barrier, 1, device_id=(nbr,),
                                device_id_type=pl.DeviceIdType.MESH)
        pl.semaphore_wait(barrier, 2)

        gather[me] = x_ref[...]
        y = jnp.dot(x_ref[...], w_ref[...],
                    preferred_element_type=jnp.float32)
        out_ref[pl.ds(me * M_PER, M_PER), :] = jnp.maximum(y, 0.0)

        for h in range(N_DEV - 1):
            src_o = _ring((r - h) % N_DEV)
            rdma = pltpu.make_async_remote_copy(
                src_ref=gather.at[src_o],
                dst_ref=gather.at[src_o],
                send_sem=send_sems.at[h],
                recv_sem=recv_sems.at[h],
                device_id=(right,),
                device_id_type=pl.DeviceIdType.MESH,
            )
            rdma.start()
            rdma.wait()
            o = _ring((r - h - 1) % N_DEV)
            y = jnp.dot(gather[o], w_ref[...],
                        preferred_element_type=jnp.float32)
            out_ref[pl.ds(o * M_PER, M_PER), :] = jnp.maximum(y, 0.0)

        amax = jnp.max(out_ref[...])
        amax_src[...] = jnp.full((1, 128), amax, jnp.float32)
        amax_buf[pl.ds(me, 1)] = jnp.full((1, 128), amax, jnp.float32)
        sends = []
        for k in range(N_DEV - 1):
            tgt = _ring((r + 1 + k) % N_DEV)
            a = pltpu.make_async_remote_copy(
                src_ref=amax_src,
                dst_ref=amax_buf.at[pl.ds(me, 1)],
                send_sem=a_send_sems.at[k],
                recv_sem=a_recv_sems.at[me],
                device_id=(tgt,),
                device_id_type=pl.DeviceIdType.MESH,
            )
            a.start()
            sends.append(a)
        for k in range(N_DEV - 1):
            src_dev = _ring((r + 1 + k) % N_DEV)
            recv = pltpu.make_async_remote_copy(
                src_ref=amax_src,
                dst_ref=amax_buf.at[pl.ds(src_dev, 1)],
                send_sem=a_send_sems.at[k],
                recv_sem=a_recv_sems.at[src_dev],
                device_id=(src_dev,),
                device_id_type=pl.DeviceIdType.MESH,
            )
            recv.wait_recv()
        for a in sends:
            a.wait_send()

        amax_g = jnp.max(amax_buf[...])
        scale = amax_g / 127.0
        vals = out_ref[...]
        q = jnp.clip(jnp.round(vals / scale), -127.0, 127.0)
        out_ref[...] = q * scale

    return pl.pallas_call(
        body,
        out_shape=jax.ShapeDtypeStruct((N_DEV * M_PER, N_PER), jnp.float32),
        in_specs=[pl.BlockSpec(memory_space=pltpu.VMEM),
                  pl.BlockSpec(memory_space=pltpu.VMEM)],
        out_specs=pl.BlockSpec(memory_space=pltpu.VMEM),
        scratch_shapes=[
            pltpu.VMEM((N_DEV, M_PER, K), jnp.bfloat16),
            pltpu.VMEM((1, 128), jnp.float32),
            pltpu.VMEM((N_DEV, 128), jnp.float32),
            pltpu.SemaphoreType.DMA((N_DEV - 1,)),
            pltpu.SemaphoreType.DMA((N_DEV - 1,)),
            pltpu.SemaphoreType.DMA((N_DEV - 1,)),
            pltpu.SemaphoreType.DMA((N_DEV,)),
        ],
        compiler_params=pltpu.CompilerParams(collective_id=0),
    )(x16, w16)


# baseline (device time: 372655 ns/iter reference)
import jax
import jax.numpy as jnp
from jax import lax
from jax.experimental import pallas as pl
from jax.experimental.pallas import tpu as pltpu

N_DEV = 8
M_PER = 512
K = 4096
N_PER = 256


def _ring(p):
    return jnp.where(p < 4, p, 11 - p)


def kernel(x, w_mat):
    x16 = x.astype(jnp.bfloat16)
    w16 = w_mat.astype(jnp.bfloat16)

    def body(x_ref, w_ref, out_ref, gather, amax_src, amax_buf,
             send_sems, recv_sems, a_send_sems, a_recv_sems):
        me = lax.axis_index("i")
        r = _ring(me)
        right = _ring((r + 1) % N_DEV)
        left = _ring((r - 1) % N_DEV)

        barrier = pltpu.get_barrier_semaphore()
        for nbr in (left, right):
            pl.semaphore_signal(barrier, 1, device_id=(nbr,),
                                device_id_type=pl.DeviceIdType.MESH)
        pl.semaphore_wait(barrier, 2)

        gather[me] = x_ref[...]
        y = jnp.dot(x_ref[...], w_ref[...],
                    preferred_element_type=jnp.float32)
        out_ref[pl.ds(me * M_PER, M_PER), :] = jnp.maximum(y, 0.0)

        for h in range(N_DEV - 1):
            src_o = _ring((r - h) % N_DEV)
            rdma = pltpu.make_async_remote_copy(
                src_ref=gather.at[src_o],
                dst_ref=gather.at[src_o],
                send_sem=send_sems.at[h],
                recv_sem=recv_sems.at[h],
                device_id=(right,),
                device_id_type=pl.DeviceIdType.MESH,
            )
            rdma.start()
            rdma.wait()
            o = _ring((r - h - 1) % N_DEV)
            y = jnp.dot(gather[o], w_ref[...],
                        preferred_element_type=jnp.float32)
            out_ref[pl.ds(o * M_PER, M_PER), :] = jnp.maximum(y, 0.0)

        amax = jnp.max(out_ref[...])
        amax_src[...] = jnp.full((1, 128), amax, jnp.float32)
        amax_buf[pl.ds(me, 1)] = jnp.full((1, 128), amax, jnp.float32)
        sends = []
        for k in range(N_DEV - 1):
            tgt = _ring((r + 1 + k) % N_DEV)
            a = pltpu.make_async_remote_copy(
                src_ref=amax_src,
                dst_ref=amax_buf.at[pl.ds(me, 1)],
                send_sem=a_send_sems.at[k],
                recv_sem=a_recv_sems.at[me],
                device_id=(tgt,),
                device_id_type=pl.DeviceIdType.MESH,
            )
            a.start()
            sends.append(a)
        for k in range(N_DEV - 1):
            src_dev = _ring((r + 1 + k) % N_DEV)
            recv = pltpu.make_async_remote_copy(
                src_ref=amax_src,
                dst_ref=amax_buf.at[pl.ds(src_dev, 1)],
                send_sem=a_send_sems.at[k],
                recv_sem=a_recv_sems.at[src_dev],
                device_id=(src_dev,),
                device_id_type=pl.DeviceIdType.MESH,
            )
            recv.wait_recv()
        for a in sends:
            a.wait_send()

        amax_g = jnp.max(amax_buf[...])
        scale = amax_g / 127.0
        vals = out_ref[...]
        q = jnp.clip(jnp.round(vals / scale), -127.0, 127.0)
        out_ref[...] = q * scale

    return pl.pallas_call(
        body,
        out_shape=jax.ShapeDtypeStruct((N_DEV * M_PER, N_PER), jnp.float32),
        in_specs=[pl.BlockSpec(memory_space=pltpu.VMEM),
                  pl.BlockSpec(memory_space=pltpu.VMEM)],
        out_specs=pl.BlockSpec(memory_space=pltpu.VMEM),
        scratch_shapes=[
            pltpu.VMEM((N_DEV, M_PER, K), jnp.bfloat16),
            pltpu.VMEM((1, 128), jnp.float32),
            pltpu.VMEM((N_DEV, 128), jnp.float32),
            pltpu.SemaphoreType.DMA((N_DEV - 1,)),
            pltpu.SemaphoreType.DMA((N_DEV - 1,)),
            pltpu.SemaphoreType.DMA((N_DEV - 1,)),
            pltpu.SemaphoreType.DMA((N_DEV,)),
        ],
        compiler_params=pltpu.CompilerParams(
            collective_id=0, vmem_limit_bytes=100 * 1024 * 1024),
    )(x16, w16)


# device time: 198335 ns/iter; 1.8789x vs baseline; 1.8789x over previous
import jax
import jax.numpy as jnp
from jax import lax
from jax.experimental import pallas as pl
from jax.experimental.pallas import tpu as pltpu

N_DEV = 8
M_PER = 512
K = 4096
N_PER = 256


def _ring(p):
    return jnp.where(p < 4, p, 11 - p)


def kernel(x, w_mat):
    x16 = x.astype(jnp.bfloat16)
    w16 = w_mat.astype(jnp.bfloat16)

    H = M_PER // 2

    def body(x_ref, w_ref, out_ref, gather, amax_src, amax_buf,
             fsend_sems, frecv_sems, bsend_sems, brecv_sems,
             a_send_sems, a_recv_sems):
        me = lax.axis_index("i")
        r = _ring(me)
        right = _ring((r + 1) % N_DEV)
        left = _ring((r - 1) % N_DEV)

        barrier = pltpu.get_barrier_semaphore()
        for nbr in (left, right):
            pl.semaphore_signal(barrier, 1, device_id=(nbr,),
                                device_id_type=pl.DeviceIdType.MESH)
        pl.semaphore_wait(barrier, 2)

        gather[me] = x_ref[...]

        def gemm_rows(o, row0, nrows):
            blk = gather[o, pl.ds(row0, nrows)]
            y = jnp.dot(blk, w_ref[...], preferred_element_type=jnp.float32)
            out_ref[pl.ds(o * M_PER + row0, nrows), :] = jnp.maximum(y, 0.0)

        def start_hop(h):
            f_o = _ring((r - h) % N_DEV)
            fwd = pltpu.make_async_remote_copy(
                src_ref=gather.at[f_o, pl.ds(0, H)],
                dst_ref=gather.at[f_o, pl.ds(0, H)],
                send_sem=fsend_sems.at[h],
                recv_sem=frecv_sems.at[h],
                device_id=(right,),
                device_id_type=pl.DeviceIdType.MESH,
            )
            b_o = _ring((r + h) % N_DEV)
            bwd = pltpu.make_async_remote_copy(
                src_ref=gather.at[b_o, pl.ds(H, H)],
                dst_ref=gather.at[b_o, pl.ds(H, H)],
                send_sem=bsend_sems.at[h],
                recv_sem=brecv_sems.at[h],
                device_id=(left,),
                device_id_type=pl.DeviceIdType.MESH,
            )
            fwd.start()
            bwd.start()
            return fwd, bwd

        sends = [start_hop(0)]
        gemm_rows(me, 0, M_PER)
        for h in range(N_DEV - 1):
            fwd, bwd = sends[h]
            fwd.wait_recv()
            bwd.wait_recv()
            if h < N_DEV - 2:
                sends.append(start_hop(h + 1))
            gemm_rows(_ring((r - 1 - h) % N_DEV), 0, H)
            gemm_rows(_ring((r + 1 + h) % N_DEV), H, H)
        for fwd, bwd in sends:
            fwd.wait_send()
            bwd.wait_send()

        amax = jnp.max(out_ref[...])
        amax_src[...] = jnp.full((1, 128), amax, jnp.float32)
        amax_buf[pl.ds(me, 1)] = jnp.full((1, 128), amax, jnp.float32)
        sends = []
        for k in range(N_DEV - 1):
            tgt = _ring((r + 1 + k) % N_DEV)
            a = pltpu.make_async_remote_copy(
                src_ref=amax_src,
                dst_ref=amax_buf.at[pl.ds(me, 1)],
                send_sem=a_send_sems.at[k],
                recv_sem=a_recv_sems.at[me],
                device_id=(tgt,),
                device_id_type=pl.DeviceIdType.MESH,
            )
            a.start()
            sends.append(a)
        for k in range(N_DEV - 1):
            src_dev = _ring((r + 1 + k) % N_DEV)
            recv = pltpu.make_async_remote_copy(
                src_ref=amax_src,
                dst_ref=amax_buf.at[pl.ds(src_dev, 1)],
                send_sem=a_send_sems.at[k],
                recv_sem=a_recv_sems.at[src_dev],
                device_id=(src_dev,),
                device_id_type=pl.DeviceIdType.MESH,
            )
            recv.wait_recv()
        for a in sends:
            a.wait_send()

        amax_g = jnp.max(amax_buf[...])
        scale = amax_g / 127.0
        vals = out_ref[...]
        q = jnp.clip(jnp.round(vals / scale), -127.0, 127.0)
        out_ref[...] = q * scale

    return pl.pallas_call(
        body,
        out_shape=jax.ShapeDtypeStruct((N_DEV * M_PER, N_PER), jnp.float32),
        in_specs=[pl.BlockSpec(memory_space=pltpu.VMEM),
                  pl.BlockSpec(memory_space=pltpu.VMEM)],
        out_specs=pl.BlockSpec(memory_space=pltpu.VMEM),
        scratch_shapes=[
            pltpu.VMEM((N_DEV, M_PER, K), jnp.bfloat16),
            pltpu.VMEM((1, 128), jnp.float32),
            pltpu.VMEM((N_DEV, 128), jnp.float32),
            pltpu.SemaphoreType.DMA((N_DEV - 1,)),
            pltpu.SemaphoreType.DMA((N_DEV - 1,)),
            pltpu.SemaphoreType.DMA((N_DEV - 1,)),
            pltpu.SemaphoreType.DMA((N_DEV - 1,)),
            pltpu.SemaphoreType.DMA((N_DEV - 1,)),
            pltpu.SemaphoreType.DMA((N_DEV,)),
        ],
        compiler_params=pltpu.CompilerParams(
            collective_id=0, vmem_limit_bytes=100 * 1024 * 1024),
    )(x16, w16)


# device time: 180739 ns/iter; 2.0618x vs baseline; 1.0974x over previous
import jax
import jax.numpy as jnp
from jax import lax
from jax.experimental import pallas as pl
from jax.experimental.pallas import tpu as pltpu

N_DEV = 8
M_PER = 512
K = 4096
N_PER = 256


def _ring(p):
    return jnp.where(p < 4, p, 11 - p)


def kernel(x, w_mat):
    H = M_PER // 2
    NSEG = 2
    SEG = H // NSEG

    def body(x_ref, w_ref, out_ref, gather, w_vmem, amax_src, amax_buf,
             fsend_sems, frecv_sems, bsend_sems, brecv_sems,
             a_send_sems, a_recv_sems):
        me = lax.axis_index("i")
        r = _ring(me)
        right = _ring((r + 1) % N_DEV)
        left = _ring((r - 1) % N_DEV)

        barrier = pltpu.get_barrier_semaphore()
        for nbr in (left, right):
            pl.semaphore_signal(barrier, 1, device_id=(nbr,),
                                device_id_type=pl.DeviceIdType.MESH)
        pl.semaphore_wait(barrier, 2)

        gather[me] = x_ref[...].astype(jnp.bfloat16)
        w_vmem[...] = w_ref[...].astype(jnp.bfloat16)

        amax_blocks = []

        def gemm_rows(o, row0, nrows):
            blk = gather[o, pl.ds(row0, nrows)]
            y = jnp.dot(blk, w_vmem[...], preferred_element_type=jnp.float32)
            y = jnp.maximum(y, 0.0)
            amax_blocks.append(jnp.max(y))
            out_ref[pl.ds(o * M_PER + row0, nrows), :] = y

        def start_seg(h, s):
            f_o = _ring((r - h) % N_DEV)
            fwd = pltpu.make_async_remote_copy(
                src_ref=gather.at[f_o, pl.ds(s * SEG, SEG)],
                dst_ref=gather.at[f_o, pl.ds(s * SEG, SEG)],
                send_sem=fsend_sems.at[h, s],
                recv_sem=frecv_sems.at[h, s],
                device_id=(right,),
                device_id_type=pl.DeviceIdType.MESH,
            )
            b_o = _ring((r + h) % N_DEV)
            bwd = pltpu.make_async_remote_copy(
                src_ref=gather.at[b_o, pl.ds(H + s * SEG, SEG)],
                dst_ref=gather.at[b_o, pl.ds(H + s * SEG, SEG)],
                send_sem=bsend_sems.at[h, s],
                recv_sem=brecv_sems.at[h, s],
                device_id=(left,),
                device_id_type=pl.DeviceIdType.MESH,
            )
            fwd.start()
            bwd.start()
            return fwd, bwd

        sends = {(0, s): start_seg(0, s) for s in range(NSEG)}
        gemm_rows(me, 0, M_PER)
        for h in range(N_DEV - 1):
            for s in range(NSEG):
                fwd, bwd = sends[(h, s)]
                fwd.wait_recv()
                bwd.wait_recv()
                if h < N_DEV - 2:
                    sends[(h + 1, s)] = start_seg(h + 1, s)
            gemm_rows(_ring((r - 1 - h) % N_DEV), 0, H)
            gemm_rows(_ring((r + 1 + h) % N_DEV), H, H)
        for fwd, bwd in sends.values():
            fwd.wait_send()
            bwd.wait_send()

        amax = jnp.max(jnp.stack(amax_blocks))
        amax_src[...] = jnp.full((1, 128), amax, jnp.float32)
        amax_buf[pl.ds(me, 1)] = jnp.full((1, 128), amax, jnp.float32)
        sends = []
        for k in range(N_DEV - 1):
            tgt = _ring((r + 1 + k) % N_DEV)
            a = pltpu.make_async_remote_copy(
                src_ref=amax_src,
                dst_ref=amax_buf.at[pl.ds(me, 1)],
                send_sem=a_send_sems.at[k],
                recv_sem=a_recv_sems.at[me],
                device_id=(tgt,),
                device_id_type=pl.DeviceIdType.MESH,
            )
            a.start()
            sends.append(a)
        for k in range(N_DEV - 1):
            src_dev = _ring((r + 1 + k) % N_DEV)
            recv = pltpu.make_async_remote_copy(
                src_ref=amax_src,
                dst_ref=amax_buf.at[pl.ds(src_dev, 1)],
                send_sem=a_send_sems.at[k],
                recv_sem=a_recv_sems.at[src_dev],
                device_id=(src_dev,),
                device_id_type=pl.DeviceIdType.MESH,
            )
            recv.wait_recv()
        for a in sends:
            a.wait_send()

        amax_g = jnp.max(amax_buf[...])
        scale = amax_g / 127.0
        vals = out_ref[...]
        q = jnp.clip(jnp.round(vals / scale), -127.0, 127.0)
        out_ref[...] = q * scale

    return pl.pallas_call(
        body,
        out_shape=jax.ShapeDtypeStruct((N_DEV * M_PER, N_PER), jnp.float32),
        in_specs=[pl.BlockSpec(memory_space=pltpu.VMEM),
                  pl.BlockSpec(memory_space=pltpu.VMEM)],
        out_specs=pl.BlockSpec(memory_space=pltpu.VMEM),
        scratch_shapes=[
            pltpu.VMEM((N_DEV, M_PER, K), jnp.bfloat16),
            pltpu.VMEM((K, N_PER), jnp.bfloat16),
            pltpu.VMEM((1, 128), jnp.float32),
            pltpu.VMEM((N_DEV, 128), jnp.float32),
            pltpu.SemaphoreType.DMA((N_DEV - 1, NSEG)),
            pltpu.SemaphoreType.DMA((N_DEV - 1, NSEG)),
            pltpu.SemaphoreType.DMA((N_DEV - 1, NSEG)),
            pltpu.SemaphoreType.DMA((N_DEV - 1, NSEG)),
            pltpu.SemaphoreType.DMA((N_DEV - 1,)),
            pltpu.SemaphoreType.DMA((N_DEV,)),
        ],
        compiler_params=pltpu.CompilerParams(
            collective_id=0, vmem_limit_bytes=100 * 1024 * 1024),
    )(x, w_mat)


# device time: 159640 ns/iter; 2.3343x vs baseline; 1.1322x over previous
import jax
import jax.numpy as jnp
from jax import lax
from jax.experimental import pallas as pl
from jax.experimental.pallas import tpu as pltpu

N_DEV = 8
M_PER = 512
K = 4096
N_PER = 256


def _ring(p):
    return jnp.where(p < 4, p, 11 - p)


def kernel(x, w_mat):
    H = M_PER // 2
    SEG = 128

    def body(x_ref, w_ref, out_ref, gather, w_vmem, amax_src, amax_buf,
             fsend, frecv, bsend, brecv, csend, crecv,
             a_send_sems, a_recv_sems):
        me = lax.axis_index("i")
        r = _ring(me)
        sq = jnp.where(r < 4, 0, 4)
        rr = r - sq
        right = _ring(sq + (rr + 1) % 4)
        left = _ring(sq + (rr - 1) % 4)
        zpair = _ring(7 - r)

        barrier = pltpu.get_barrier_semaphore()
        for nbr in (left, right, zpair):
            pl.semaphore_signal(barrier, 1, device_id=(nbr,),
                                device_id_type=pl.DeviceIdType.MESH)
        pl.semaphore_wait(barrier, 3)

        gather[me] = x_ref[...].astype(jnp.bfloat16)
        w_vmem[...] = w_ref[...].astype(jnp.bfloat16)

        m = [_ring(sq + (rr - k) % 4) for k in range(4)]
        p = [_ring(7 - (sq + (rr - k) % 4)) for k in range(4)]
        mb = [_ring(sq + (rr + k) % 4) for k in range(4)]
        pb = [_ring(7 - (sq + (rr + k) % 4)) for k in range(4)]

        ft = [m[0], m[1], m[2], p[0], p[1], p[2]]
        fr = [m[1], m[2], m[3], p[1], p[2], p[3]]
        bt = [mb[0], mb[1], mb[2], pb[0], pb[1], pb[2]]
        br = [mb[1], mb[2], mb[3], pb[1], pb[2], pb[3]]

        def seg_desc(chunk, row0, ssems, rsems, t, s, dev):
            return pltpu.make_async_remote_copy(
                src_ref=gather.at[chunk, pl.ds(row0, SEG)],
                dst_ref=gather.at[chunk, pl.ds(row0, SEG)],
                send_sem=ssems.at[t, s],
                recv_sem=rsems.at[t, s],
                device_id=(dev,),
                device_id_type=pl.DeviceIdType.MESH,
            )

        sends = []

        def fgo(t, s):
            d = seg_desc(ft[t], s * SEG, fsend, frecv, t, s, right)
            d.start()
            sends.append(d)

        def bgo(t, s):
            d = seg_desc(bt[t], H + s * SEG, bsend, brecv, t, s, left)
            d.start()
            sends.append(d)

        def fwait(t, s):
            seg_desc(fr[t], s * SEG, fsend, frecv, t, s, right).wait_recv()

        def bwait(t, s):
            seg_desc(br[t], H + s * SEG, bsend, brecv, t, s, left).wait_recv()

        def cross_desc(chunk, s):
            return pltpu.make_async_remote_copy(
                src_ref=gather.at[chunk, pl.ds(s * SEG, SEG)],
                dst_ref=gather.at[chunk, pl.ds(s * SEG, SEG)],
                send_sem=csend.at[s],
                recv_sem=crecv.at[s],
                device_id=(zpair,),
                device_id_type=pl.DeviceIdType.MESH,
            )

        amax_blocks = []

        def gemm_rows(o, row0, nrows):
            blk = gather[o, pl.ds(row0, nrows)]
            y = jnp.dot(blk, w_vmem[...], preferred_element_type=jnp.float32)
            y = jnp.maximum(y, 0.0)
            amax_blocks.append(jnp.max(y))
            out_ref[pl.ds(o * M_PER + row0, nrows), :] = y

        for s in (0, 1):
            fgo(0, s)
        for s in (0, 1):
            bgo(0, s)
        for s in range(4):
            d = cross_desc(me, s)
            d.start()
            sends.append(d)
        gemm_rows(me, 0, M_PER)

        for s in (0, 1):
            fwait(0, s)
            fgo(1, s)
        for s in (0, 1):
            bwait(0, s)
            bgo(1, s)
        gemm_rows(m[1], 0, H)
        gemm_rows(mb[1], H, H)

        for s in (0, 1):
            cross_desc(zpair, s).wait_recv()
            fgo(3, s)
        for s in (2, 3):
            cross_desc(zpair, s).wait_recv()
            bgo(3, s - 2)
        gemm_rows(zpair, 0, M_PER)

        for s in (0, 1):
            fwait(1, s)
            fgo(2, s)
        for s in (0, 1):
            bwait(1, s)
            bgo(2, s)
        gemm_rows(m[2], 0, H)
        gemm_rows(mb[2], H, H)

        for s in (0, 1):
            fwait(3, s)
            fgo(4, s)
        for s in (0, 1):
            bwait(3, s)
            bgo(4, s)
        gemm_rows(p[1], 0, H)
        gemm_rows(pb[1], H, H)

        for s in (0, 1):
            fwait(2, s)
        for s in (0, 1):
            bwait(2, s)
        gemm_rows(m[3], 0, H)
        gemm_rows(mb[3], H, H)

        for s in (0, 1):
            fwait(4, s)
            fgo(5, s)
        for s in (0, 1):
            bwait(4, s)
            bgo(5, s)
        gemm_rows(p[2], 0, H)
        gemm_rows(pb[2], H, H)

        for s in (0, 1):
            fwait(5, s)
        for s in (0, 1):
            bwait(5, s)
        gemm_rows(p[3], 0, H)
        gemm_rows(pb[3], H, H)

        for d in sends:
            d.wait_send()

        amax = jnp.max(jnp.stack(amax_blocks))
        amax_src[...] = jnp.full((1, 128), amax, jnp.float32)
        amax_buf[pl.ds(me, 1)] = jnp.full((1, 128), amax, jnp.float32)
        a_sends = []
        for k in range(N_DEV - 1):
            tgt = _ring((r + 1 + k) % N_DEV)
            a = pltpu.make_async_remote_copy(
                src_ref=amax_src,
                dst_ref=amax_buf.at[pl.ds(me, 1)],
                send_sem=a_send_sems.at[k],
                recv_sem=a_recv_sems.at[me],
                device_id=(tgt,),
                device_id_type=pl.DeviceIdType.MESH,
            )
            a.start()
            a_sends.append(a)
        for k in range(N_DEV - 1):
            src_dev = _ring((r + 1 + k) % N_DEV)
            recv = pltpu.make_async_remote_copy(
                src_ref=amax_src,
                dst_ref=amax_buf.at[pl.ds(src_dev, 1)],
                send_sem=a_send_sems.at[k],
                recv_sem=a_recv_sems.at[src_dev],
                device_id=(src_dev,),
                device_id_type=pl.DeviceIdType.MESH,
            )
            recv.wait_recv()
        for a in a_sends:
            a.wait_send()

        amax_g = jnp.max(amax_buf[...])
        scale = amax_g / 127.0
        vals = out_ref[...]
        q = jnp.clip(jnp.round(vals / scale), -127.0, 127.0)
        out_ref[...] = q * scale

    return pl.pallas_call(
        body,
        out_shape=jax.ShapeDtypeStruct((N_DEV * M_PER, N_PER), jnp.float32),
        in_specs=[pl.BlockSpec(memory_space=pltpu.VMEM),
                  pl.BlockSpec(memory_space=pltpu.VMEM)],
        out_specs=pl.BlockSpec(memory_space=pltpu.VMEM),
        scratch_shapes=[
            pltpu.VMEM((N_DEV, M_PER, K), jnp.bfloat16),
            pltpu.VMEM((K, N_PER), jnp.bfloat16),
            pltpu.VMEM((1, 128), jnp.float32),
            pltpu.VMEM((N_DEV, 128), jnp.float32),
            pltpu.SemaphoreType.DMA((6, 2)),
            pltpu.SemaphoreType.DMA((6, 2)),
            pltpu.SemaphoreType.DMA((6, 2)),
            pltpu.SemaphoreType.DMA((6, 2)),
            pltpu.SemaphoreType.DMA((4,)),
            pltpu.SemaphoreType.DMA((4,)),
            pltpu.SemaphoreType.DMA((N_DEV - 1,)),
            pltpu.SemaphoreType.DMA((N_DEV,)),
        ],
        compiler_params=pltpu.CompilerParams(
            collective_id=0, vmem_limit_bytes=100 * 1024 * 1024),
    )(x, w_mat)


# device time: 137138 ns/iter; 2.7174x vs baseline; 1.1641x over previous
import jax
import jax.numpy as jnp
from jax import lax
from jax.experimental import pallas as pl
from jax.experimental.pallas import tpu as pltpu

N_DEV = 8
M_PER = 512
K = 4096
N_PER = 256


def _ring(p):
    return jnp.where(p < 4, p, 11 - p)


def kernel(x, w_mat):
    H = M_PER // 2
    SEG = 128

    def body(x_ref, w_ref, out_ref, gather, w_vmem, amax_src, amax_buf,
             fsend, frecv, bsend, brecv, csend, crecv,
             a_send_sems, a_recv_sems):
        me = lax.axis_index("i")
        r = _ring(me)
        sq = jnp.where(r < 4, 0, 4)
        rr = r - sq
        right = _ring(sq + (rr + 1) % 4)
        left = _ring(sq + (rr - 1) % 4)
        zpair = _ring(7 - r)

        barrier = pltpu.get_barrier_semaphore()
        for nbr in (left, right, zpair):
            pl.semaphore_signal(barrier, 1, device_id=(nbr,),
                                device_id_type=pl.DeviceIdType.MESH)
        pl.semaphore_wait(barrier, 3)

        gather[me] = x_ref[...].astype(jnp.bfloat16)
        w_vmem[...] = w_ref[...].astype(jnp.bfloat16)

        m = [_ring(sq + (rr - k) % 4) for k in range(4)]
        p = [_ring(7 - (sq + (rr - k) % 4)) for k in range(4)]
        mb = [_ring(sq + (rr + k) % 4) for k in range(4)]
        pb = [_ring(7 - (sq + (rr + k) % 4)) for k in range(4)]

        ft = [m[0], m[1], m[2], p[0], p[1]]
        fr = [m[1], m[2], m[3], p[1], p[2]]
        bt = [mb[0], mb[1], mb[2], pb[0], pb[1]]
        br = [mb[1], mb[2], mb[3], pb[1], pb[2]]

        def seg_desc(chunk, row0, ssems, rsems, t, s, dev):
            return pltpu.make_async_remote_copy(
                src_ref=gather.at[chunk, pl.ds(row0, SEG)],
                dst_ref=gather.at[chunk, pl.ds(row0, SEG)],
                send_sem=ssems.at[t, s],
                recv_sem=rsems.at[t, s],
                device_id=(dev,),
                device_id_type=pl.DeviceIdType.MESH,
            )

        sends = []

        def fgo(t, s):
            d = seg_desc(ft[t], s * SEG, fsend, frecv, t, s, right)
            d.start()
            sends.append(d)

        def bgo(t, s):
            d = seg_desc(bt[t], H + s * SEG, bsend, brecv, t, s, left)
            d.start()
            sends.append(d)

        def fwait(t, s):
            seg_desc(fr[t], s * SEG, fsend, frecv, t, s, right).wait_recv()

        def bwait(t, s):
            seg_desc(br[t], H + s * SEG, bsend, brecv, t, s, left).wait_recv()

        def cross_desc(chunk, row0, t):
            return pltpu.make_async_remote_copy(
                src_ref=gather.at[chunk, pl.ds(row0, SEG)],
                dst_ref=gather.at[chunk, pl.ds(row0, SEG)],
                send_sem=csend.at[t],
                recv_sem=crecv.at[t],
                device_id=(zpair,),
                device_id_type=pl.DeviceIdType.MESH,
            )

        def cgo(chunk, row0, t):
            d = cross_desc(chunk, row0, t)
            d.start()
            sends.append(d)

        amax_blocks = []

        def gemm_rows(o, row0, nrows):
            blk = gather[o, pl.ds(row0, nrows)]
            y = jnp.dot(blk, w_vmem[...], preferred_element_type=jnp.float32)
            y = jnp.maximum(y, 0.0)
            amax_blocks.append(jnp.max(y))
            out_ref[pl.ds(o * M_PER + row0, nrows), :] = y

        for s in (0, 1):
            fgo(0, s)
        for s in (0, 1):
            bgo(0, s)
        for s in range(4):
            cgo(me, s * SEG, s)
        gemm_rows(me, 0, M_PER)

        for s in (0, 1):
            fwait(0, s)
            fgo(1, s)
            cgo(m[1], s * SEG, 4 + s)
        for s in (0, 1):
            bwait(0, s)
            bgo(1, s)
            cgo(mb[1], H + s * SEG, 6 + s)
        gemm_rows(m[1], 0, H)
        gemm_rows(mb[1], H, H)

        for s in (0, 1):
            cross_desc(zpair, s * SEG, s).wait_recv()
            fgo(3, s)
        for s in (2, 3):
            cross_desc(zpair, s * SEG, s).wait_recv()
            bgo(3, s - 2)
        gemm_rows(zpair, 0, M_PER)

        for s in (0, 1):
            fwait(1, s)
            fgo(2, s)
        for s in (0, 1):
            bwait(1, s)
            bgo(2, s)
        gemm_rows(m[2], 0, H)
        gemm_rows(mb[2], H, H)

        for s in (0, 1):
            fwait(3, s)
            fgo(4, s)
        for s in (0, 1):
            bwait(3, s)
            bgo(4, s)
        gemm_rows(p[1], 0, H)
        gemm_rows(pb[1], H, H)

        for s in (0, 1):
            fwait(2, s)
        for s in (0, 1):
            bwait(2, s)
        gemm_rows(m[3], 0, H)
        gemm_rows(mb[3], H, H)

        for s in (0, 1):
            cross_desc(p[3], s * SEG, 4 + s).wait_recv()
        gemm_rows(p[3], 0, H)
        for s in (0, 1):
            cross_desc(pb[3], H + s * SEG, 6 + s).wait_recv()
        gemm_rows(pb[3], H, H)

        for s in (0, 1):
            fwait(4, s)
        for s in (0, 1):
            bwait(4, s)
        gemm_rows(p[2], 0, H)
        gemm_rows(pb[2], H, H)

        for d in sends:
            d.wait_send()

        amax = jnp.max(jnp.stack(amax_blocks))
        amax_src[...] = jnp.full((1, 128), amax, jnp.float32)
        amax_buf[pl.ds(me, 1)] = jnp.full((1, 128), amax, jnp.float32)
        a_sends = []
        for k in range(N_DEV - 1):
            tgt = _ring((r + 1 + k) % N_DEV)
            a = pltpu.make_async_remote_copy(
                src_ref=amax_src,
                dst_ref=amax_buf.at[pl.ds(me, 1)],
                send_sem=a_send_sems.at[k],
                recv_sem=a_recv_sems.at[me],
                device_id=(tgt,),
                device_id_type=pl.DeviceIdType.MESH,
            )
            a.start()
            a_sends.append(a)
        for k in range(N_DEV - 1):
            src_dev = _ring((r + 1 + k) % N_DEV)
            recv = pltpu.make_async_remote_copy(
                src_ref=amax_src,
                dst_ref=amax_buf.at[pl.ds(src_dev, 1)],
                send_sem=a_send_sems.at[k],
                recv_sem=a_recv_sems.at[src_dev],
                device_id=(src_dev,),
                device_id_type=pl.DeviceIdType.MESH,
            )
            recv.wait_recv()
        for a in a_sends:
            a.wait_send()

        amax_g = jnp.max(amax_buf[...])
        scale = amax_g / 127.0
        vals = out_ref[...]
        q = jnp.clip(jnp.round(vals / scale), -127.0, 127.0)
        out_ref[...] = q * scale

    return pl.pallas_call(
        body,
        out_shape=jax.ShapeDtypeStruct((N_DEV * M_PER, N_PER), jnp.float32),
        in_specs=[pl.BlockSpec(memory_space=pltpu.VMEM),
                  pl.BlockSpec(memory_space=pltpu.VMEM)],
        out_specs=pl.BlockSpec(memory_space=pltpu.VMEM),
        scratch_shapes=[
            pltpu.VMEM((N_DEV, M_PER, K), jnp.bfloat16),
            pltpu.VMEM((K, N_PER), jnp.bfloat16),
            pltpu.VMEM((1, 128), jnp.float32),
            pltpu.VMEM((N_DEV, 128), jnp.float32),
            pltpu.SemaphoreType.DMA((5, 2)),
            pltpu.SemaphoreType.DMA((5, 2)),
            pltpu.SemaphoreType.DMA((5, 2)),
            pltpu.SemaphoreType.DMA((5, 2)),
            pltpu.SemaphoreType.DMA((8,)),
            pltpu.SemaphoreType.DMA((8,)),
            pltpu.SemaphoreType.DMA((N_DEV - 1,)),
            pltpu.SemaphoreType.DMA((N_DEV,)),
        ],
        compiler_params=pltpu.CompilerParams(
            collective_id=0, vmem_limit_bytes=100 * 1024 * 1024),
    )(x, w_mat)


# device time: 130136 ns/iter; 2.8636x vs baseline; 1.0538x over previous
import jax
import jax.numpy as jnp
from jax import lax
from jax.experimental import pallas as pl
from jax.experimental.pallas import tpu as pltpu

N_DEV = 8
M_PER = 512
K = 4096
N_PER = 256


def _ring(p):
    return jnp.where(p < 4, p, 11 - p)


def kernel(x, w_mat):
    H = M_PER // 2
    SEG = 128
    RT = 160
    TAIL = H - RT

    def body(x_ref, w_ref, out_ref, gather, w_vmem, amax_src, amax_buf,
             fsend, frecv, bsend, brecv, csend, crecv,
             a_send_sems, a_recv_sems):
        me = lax.axis_index("i")
        r = _ring(me)
        sq = jnp.where(r < 4, 0, 4)
        rr = r - sq
        right = _ring(sq + (rr + 1) % 4)
        left = _ring(sq + (rr - 1) % 4)
        zpair = _ring(7 - r)

        barrier = pltpu.get_barrier_semaphore()
        for nbr in (left, right, zpair):
            pl.semaphore_signal(barrier, 1, device_id=(nbr,),
                                device_id_type=pl.DeviceIdType.MESH)
        pl.semaphore_wait(barrier, 3)

        gather[me] = x_ref[...].astype(jnp.bfloat16)
        w_vmem[...] = w_ref[...].astype(jnp.bfloat16)

        m = [_ring(sq + (rr - k) % 4) for k in range(4)]
        p = [_ring(7 - (sq + (rr - k) % 4)) for k in range(4)]
        mb = [_ring(sq + (rr + k) % 4) for k in range(4)]
        pb = [_ring(7 - (sq + (rr + k) % 4)) for k in range(4)]

        ft = [m[0], m[1], m[2], p[0], p[1]]
        fr = [m[1], m[2], m[3], p[1], p[2]]
        bt = [mb[0], mb[1], mb[2], pb[0], pb[1]]
        br = [mb[1], mb[2], mb[3], pb[1], pb[2]]

        def seg_desc(chunk, row0, nrows, ssems, rsems, t, s, dev):
            return pltpu.make_async_remote_copy(
                src_ref=gather.at[chunk, pl.ds(row0, nrows)],
                dst_ref=gather.at[chunk, pl.ds(row0, nrows)],
                send_sem=ssems.at[t, s],
                recv_sem=rsems.at[t, s],
                device_id=(dev,),
                device_id_type=pl.DeviceIdType.MESH,
            )

        sends = []

        def fgo(t, s, row0=None, nrows=SEG):
            row0 = s * SEG if row0 is None else row0
            d = seg_desc(ft[t], row0, nrows, fsend, frecv, t, s, right)
            d.start()
            sends.append(d)

        def bgo(t, s, row0=None, nrows=SEG):
            row0 = H + s * SEG if row0 is None else row0
            d = seg_desc(bt[t], row0, nrows, bsend, brecv, t, s, left)
            d.start()
            sends.append(d)

        def fwait(t, s, row0=None, nrows=SEG):
            row0 = s * SEG if row0 is None else row0
            seg_desc(fr[t], row0, nrows, fsend, frecv, t, s, right).wait_recv()

        def bwait(t, s, row0=None, nrows=SEG):
            row0 = H + s * SEG if row0 is None else row0
            seg_desc(br[t], row0, nrows, bsend, brecv, t, s, left).wait_recv()

        def cross_desc(chunk, row0, t, nrows=SEG):
            return pltpu.make_async_remote_copy(
                src_ref=gather.at[chunk, pl.ds(row0, nrows)],
                dst_ref=gather.at[chunk, pl.ds(row0, nrows)],
                send_sem=csend.at[t],
                recv_sem=crecv.at[t],
                device_id=(zpair,),
                device_id_type=pl.DeviceIdType.MESH,
            )

        def cgo(chunk, row0, t, nrows=SEG):
            d = cross_desc(chunk, row0, t, nrows)
            d.start()
            sends.append(d)

        amax_blocks = []

        def gemm_rows(o, row0, nrows):
            blk = gather[o, pl.ds(row0, nrows)]
            y = jnp.dot(blk, w_vmem[...], preferred_element_type=jnp.float32)
            y = jnp.maximum(y, 0.0)
            amax_blocks.append(jnp.max(y))
            out_ref[pl.ds(o * M_PER + row0, nrows), :] = y

        for s in (0, 1):
            fgo(0, s)
        for s in (0, 1):
            bgo(0, s)
        for s in range(4):
            cgo(me, s * SEG, s)
        gemm_rows(me, 0, M_PER)

        for s in (0, 1):
            fwait(0, s)
            fgo(1, s)
            cgo(m[1], s * SEG, 4 + s)
        for s in (0, 1):
            bwait(0, s)
            bgo(1, s)
            cgo(mb[1], H + s * SEG, 6 + s)
        gemm_rows(m[1], 0, H)
        gemm_rows(mb[1], H, H)

        for s in (0, 1):
            cross_desc(zpair, s * SEG, s).wait_recv()
            fgo(3, s)
        for s in (2, 3):
            cross_desc(zpair, s * SEG, s).wait_recv()
            bgo(3, s - 2)
        gemm_rows(zpair, 0, M_PER)

        for s in (0, 1):
            fwait(1, s)
            fgo(2, s)
        cgo(m[2], RT, 8, TAIL)
        for s in (0, 1):
            bwait(1, s)
            bgo(2, s)
        cgo(mb[2], H + RT, 9, TAIL)
        gemm_rows(m[2], 0, H)
        gemm_rows(mb[2], H, H)

        for s in (0, 1):
            fwait(3, s)
        fgo(4, 0, 0, RT)
        for s in (0, 1):
            bwait(3, s)
        bgo(4, 0, H, RT)
        gemm_rows(p[1], 0, H)
        gemm_rows(pb[1], H, H)

        for s in (0, 1):
            fwait(2, s)
        for s in (0, 1):
            bwait(2, s)
        gemm_rows(m[3], 0, H)
        gemm_rows(mb[3], H, H)

        for s in (0, 1):
            cross_desc(p[3], s * SEG, 4 + s).wait_recv()
        gemm_rows(p[3], 0, H)
        for s in (0, 1):
            cross_desc(pb[3], H + s * SEG, 6 + s).wait_recv()
        gemm_rows(pb[3], H, H)

        fwait(4, 0, 0, RT)
        cross_desc(p[2], RT, 8, TAIL).wait_recv()
        gemm_rows(p[2], 0, H)
        bwait(4, 0, H, RT)
        cross_desc(pb[2], H + RT, 9, TAIL).wait_recv()
        gemm_rows(pb[2], H, H)

        for d in sends:
            d.wait_send()

        amax = jnp.max(jnp.stack(amax_blocks))
        amax_src[...] = jnp.full((1, 128), amax, jnp.float32)
        amax_buf[pl.ds(me, 1)] = jnp.full((1, 128), amax, jnp.float32)
        a_sends = []
        for k in range(N_DEV - 1):
            tgt = _ring((r + 1 + k) % N_DEV)
            a = pltpu.make_async_remote_copy(
                src_ref=amax_src,
                dst_ref=amax_buf.at[pl.ds(me, 1)],
                send_sem=a_send_sems.at[k],
                recv_sem=a_recv_sems.at[me],
                device_id=(tgt,),
                device_id_type=pl.DeviceIdType.MESH,
            )
            a.start()
            a_sends.append(a)
        for k in range(N_DEV - 1):
            src_dev = _ring((r + 1 + k) % N_DEV)
            recv = pltpu.make_async_remote_copy(
                src_ref=amax_src,
                dst_ref=amax_buf.at[pl.ds(src_dev, 1)],
                send_sem=a_send_sems.at[k],
                recv_sem=a_recv_sems.at[src_dev],
                device_id=(src_dev,),
                device_id_type=pl.DeviceIdType.MESH,
            )
            recv.wait_recv()
        for a in a_sends:
            a.wait_send()

        amax_g = jnp.max(amax_buf[...])
        scale = amax_g / 127.0
        vals = out_ref[...]
        q = jnp.clip(jnp.round(vals / scale), -127.0, 127.0)
        out_ref[...] = q * scale

    return pl.pallas_call(
        body,
        out_shape=jax.ShapeDtypeStruct((N_DEV * M_PER, N_PER), jnp.float32),
        in_specs=[pl.BlockSpec(memory_space=pltpu.VMEM),
                  pl.BlockSpec(memory_space=pltpu.VMEM)],
        out_specs=pl.BlockSpec(memory_space=pltpu.VMEM),
        scratch_shapes=[
            pltpu.VMEM((N_DEV, M_PER, K), jnp.bfloat16),
            pltpu.VMEM((K, N_PER), jnp.bfloat16),
            pltpu.VMEM((1, 128), jnp.float32),
            pltpu.VMEM((N_DEV, 128), jnp.float32),
            pltpu.SemaphoreType.DMA((5, 2)),
            pltpu.SemaphoreType.DMA((5, 2)),
            pltpu.SemaphoreType.DMA((5, 2)),
            pltpu.SemaphoreType.DMA((5, 2)),
            pltpu.SemaphoreType.DMA((10,)),
            pltpu.SemaphoreType.DMA((10,)),
            pltpu.SemaphoreType.DMA((N_DEV - 1,)),
            pltpu.SemaphoreType.DMA((N_DEV,)),
        ],
        compiler_params=pltpu.CompilerParams(
            collective_id=0, vmem_limit_bytes=100 * 1024 * 1024),
    )(x, w_mat)
